# baseline (speedup 1.0000x reference)
import functools
import math

import jax
import jax.numpy as jnp
from jax import lax
from jax.experimental import pallas as pl
from jax.experimental.pallas import tpu as pltpu
from jax.experimental.pallas import tpu_sc as plsc

D_MODEL = 64
SCALE = math.sqrt(D_MODEL)
NUM_WORKERS = 32
LANES = 16


def _emb_kernel_body(b_per_w, chunk, n_chunks,
                     x_hbm, table_hbm, out_hbm, idx_v, rows_v, sem):
    wid = lax.axis_index("s") * 2 + lax.axis_index("c")
    base = wid * b_per_w
    pltpu.sync_copy(x_hbm.at[pl.ds(base, b_per_w)], idx_v)

    def chunk_body(g, carry):
        off = g * chunk
        pltpu.async_copy(
            table_hbm.at[idx_v.at[pl.ds(off, chunk)]], rows_v, sem
        ).wait()

        def scale_body(r, c):
            for j in range(D_MODEL // LANES):
                sl = (r, pl.ds(j * LANES, LANES))
                rows_v[sl] = rows_v[sl] * SCALE
            return c

        lax.fori_loop(0, chunk, scale_body, 0, unroll=4)
        pltpu.sync_copy(rows_v, out_hbm.at[pl.ds(base + off, chunk)])
        return carry

    lax.fori_loop(0, n_chunks, chunk_body, 0)


def kernel(x, table):
    orig_shape = x.shape
    n = x.size
    xf = x.reshape(n)
    if xf.dtype != jnp.int32:
        xf = xf.astype(jnp.int32)

    b_per_w = n // NUM_WORKERS
    chunk = 512
    n_chunks = b_per_w // chunk

    mesh = plsc.VectorSubcoreMesh(core_axis_name="c", subcore_axis_name="s")
    emb = functools.partial(
        pl.kernel,
        mesh=mesh,
        compiler_params=pltpu.CompilerParams(use_tc_tiling_on_sc=False),
        out_type=jax.ShapeDtypeStruct((n, D_MODEL), jnp.float32),
        scratch_types=[
            pltpu.VMEM((b_per_w,), jnp.int32),
            pltpu.VMEM((chunk, D_MODEL), jnp.float32),
            pltpu.SemaphoreType.DMA,
        ],
    )(functools.partial(_emb_kernel_body, b_per_w, chunk, n_chunks))

    out = emb(xf, table)
    return out.reshape(*orig_shape, D_MODEL)


# speedup vs baseline: 1.0680x; 1.0680x over previous
import functools
import math

import jax
import jax.numpy as jnp
from jax import lax
from jax.experimental import pallas as pl
from jax.experimental.pallas import tpu as pltpu
from jax.experimental.pallas import tpu_sc as plsc

D_MODEL = 64
SCALE = math.sqrt(D_MODEL)
NUM_WORKERS = 32
LANES = 16
CHUNK = 512


def _scale_chunk(rows):
    def scale_body(r, c):
        for j in range(D_MODEL // LANES):
            sl = (r, pl.ds(j * LANES, LANES))
            rows[sl] = rows[sl] * SCALE
        return c

    lax.fori_loop(0, CHUNK, scale_body, 0, unroll=8)


def _emb_kernel_body(b_per_w, n_chunks,
                     x_hbm, table_hbm, out_hbm, idx_v, b0, b1,
                     gsem0, gsem1, ssem0, ssem1):
    wid = lax.axis_index("s") * 2 + lax.axis_index("c")
    base = wid * b_per_w
    pltpu.sync_copy(x_hbm.at[pl.ds(base, b_per_w)], idx_v)

    def idx_at(g):
        return idx_v.at[pl.ds(g * CHUNK, CHUNK)]

    def out_at(g):
        return out_hbm.at[pl.ds(base + g * CHUNK, CHUNK)]

    def gather(g, buf, sem):
        pltpu.async_copy(table_hbm.at[idx_at(g)], buf, sem)

    def wait_gather(buf, sem):
        pltpu.make_async_copy(table_hbm.at[idx_at(0)], buf, sem).wait()

    def scatter(g, buf, sem):
        pltpu.async_copy(buf, out_at(g), sem)

    def wait_scatter(buf, sem):
        pltpu.make_async_copy(buf, out_at(0), sem).wait()

    gather(0, b0, gsem0)

    wait_gather(b0, gsem0)
    gather(1, b1, gsem1)
    _scale_chunk(b0)
    scatter(0, b0, ssem0)

    wait_gather(b1, gsem1)
    wait_scatter(b0, ssem0)
    gather(2, b0, gsem0)
    _scale_chunk(b1)
    scatter(1, b1, ssem1)

    def pair_body(gp, carry):
        g = 2 * gp
        wait_gather(b0, gsem0)
        wait_scatter(b1, ssem1)
        gather(g + 1, b1, gsem1)
        _scale_chunk(b0)
        scatter(g, b0, ssem0)

        wait_gather(b1, gsem1)
        wait_scatter(b0, ssem0)
        gather(g + 2, b0, gsem0)
        _scale_chunk(b1)
        scatter(g + 1, b1, ssem1)
        return carry

    lax.fori_loop(1, n_chunks // 2 - 1, pair_body, 0)

    g = n_chunks - 2
    wait_gather(b0, gsem0)
    wait_scatter(b1, ssem1)
    gather(g + 1, b1, gsem1)
    _scale_chunk(b0)
    scatter(g, b0, ssem0)

    wait_gather(b1, gsem1)
    wait_scatter(b0, ssem0)
    _scale_chunk(b1)
    scatter(g + 1, b1, ssem1)
    wait_scatter(b1, ssem1)


def kernel(x, table):
    orig_shape = x.shape
    n = x.size
    xf = x.reshape(n)
    if xf.dtype != jnp.int32:
        xf = xf.astype(jnp.int32)

    b_per_w = n // NUM_WORKERS
    n_chunks = b_per_w // CHUNK

    mesh = plsc.VectorSubcoreMesh(core_axis_name="c", subcore_axis_name="s")
    emb = functools.partial(
        pl.kernel,
        mesh=mesh,
        compiler_params=pltpu.CompilerParams(use_tc_tiling_on_sc=False),
        out_type=jax.ShapeDtypeStruct((n, D_MODEL), jnp.float32),
        scratch_types=[
            pltpu.VMEM((b_per_w,), jnp.int32),
            pltpu.VMEM((CHUNK, D_MODEL), jnp.float32),
            pltpu.VMEM((CHUNK, D_MODEL), jnp.float32),
            pltpu.SemaphoreType.DMA,
            pltpu.SemaphoreType.DMA,
            pltpu.SemaphoreType.DMA,
            pltpu.SemaphoreType.DMA,
        ],
    )(functools.partial(_emb_kernel_body, b_per_w, n_chunks))

    out = emb(xf, table)
    return out.reshape(*orig_shape, D_MODEL)
